# baseline (device time: 765472 ns/iter reference)
import jax
import jax.numpy as jnp
from jax import lax
from jax.experimental import pallas as pl
from jax.experimental.pallas import tpu as pltpu

N_DEV = 16


def kernel(x, w_mat, scale_x, scale_w):
    M, k_per = x.shape
    N = w_mat.shape[1]
    m_per = M // N_DEV

    x_bf = x.astype(jnp.bfloat16)
    w_bf = w_mat.astype(jnp.bfloat16)
    scale = (scale_x * scale_w).astype(jnp.float32)

    def body(x_ref, w_ref, scale_ref, out_ref,
             comm_ref, partial_ref, send_sems, recv_sems, credit_sem):
        my = lax.axis_index("i")
        left = (my + N_DEV - 1) % N_DEV
        right = (my + 1) % N_DEV

        barrier_sem = pltpu.get_barrier_semaphore()
        for nbr in (left, right):
            pl.semaphore_signal(barrier_sem, inc=1, device_id=(nbr,),
                                device_id_type=pl.DeviceIdType.MESH)
        pl.semaphore_wait(barrier_sem, 2)

        def partial_chunk(c):
            xc = x_ref[pl.ds(c * m_per, m_per), :]
            return lax.dot_general(xc, w_ref[...], (((1,), (0,)), ((), ())),
                                   preferred_element_type=jnp.float32)

        comm_ref[0, :, :] = partial_chunk((my + N_DEV - 1) % N_DEV).astype(
            jnp.bfloat16)

        for s in range(N_DEV - 1):
            send_slot = s % 2
            recv_slot = (s + 1) % 2
            if s >= 1:
                pl.semaphore_wait(credit_sem, 1)
            rdma = pltpu.make_async_remote_copy(
                src_ref=comm_ref.at[send_slot],
                dst_ref=comm_ref.at[recv_slot],
                send_sem=send_sems.at[send_slot],
                recv_sem=recv_sems.at[recv_slot],
                device_id=(right,),
                device_id_type=pl.DeviceIdType.MESH,
            )
            rdma.start()
            c = (my + 2 * N_DEV - s - 2) % N_DEV
            partial_ref[...] = partial_chunk(c)
            rdma.wait()
            if s < N_DEV - 2:
                pl.semaphore_signal(credit_sem, inc=1, device_id=(left,),
                                    device_id_type=pl.DeviceIdType.MESH)
                comm_ref[recv_slot, :, :] = (
                    comm_ref[recv_slot, :, :].astype(jnp.float32)
                    + partial_ref[...]).astype(jnp.bfloat16)
            else:
                out_ref[...] = (comm_ref[recv_slot, :, :].astype(jnp.float32)
                                + partial_ref[...]) * scale_ref[0]

    return pl.pallas_call(
        body,
        out_shape=jax.ShapeDtypeStruct((m_per, N), jnp.float32),
        in_specs=[
            pl.BlockSpec(memory_space=pltpu.VMEM),
            pl.BlockSpec(memory_space=pltpu.VMEM),
            pl.BlockSpec(memory_space=pltpu.SMEM),
        ],
        out_specs=pl.BlockSpec(memory_space=pltpu.VMEM),
        scratch_shapes=[
            pltpu.VMEM((2, m_per, N), jnp.bfloat16),
            pltpu.VMEM((m_per, N), jnp.float32),
            pltpu.SemaphoreType.DMA((2,)),
            pltpu.SemaphoreType.DMA((2,)),
            pltpu.SemaphoreType.REGULAR,
        ],
        compiler_params=pltpu.CompilerParams(collective_id=0),
    )(x_bf, w_bf, scale)
